# Initial kernel scaffold; baseline (speedup 1.0000x reference)
import jax
import jax.numpy as jnp
from jax.experimental import pallas as pl


def kernel(spk_in, spk_thdot, spk_thdot2, slab):
    raise NotImplementedError("write your pallas kernel here")


# speedup vs baseline: 1.5186x; 1.5186x over previous
import functools

import jax
import jax.numpy as jnp
from jax.experimental import pallas as pl
from jax.experimental.pallas import tpu as pltpu

_T = 8
_N = 8
_F0 = 32
_F1 = 32
_F2 = _N
_C1 = 4
_NC1 = _N * _C1
_IN = 2 * _N
_OUT = 3
_BETA = 0.9
_THR = 1.0

def _al8(r):
    return (r + 7) // 8 * 8

_SLAB_W = 32
_R_W1 = 0
_R_W2 = _al8(_R_W1 + _IN)
_R_WTH = _al8(_R_W2 + _F0)
_R_WSP3 = _al8(_R_WTH + 2 * _N)
_R_W3 = _al8(_R_WSP3 + _F2)
_R_WRD = _al8(_R_W3 + _F1)
_R_BIAS = _al8(_R_WRD + _NC1)
_SLAB_ROWS = _al8(_R_BIAS + 5)


def _snn_body(si_ref, td_ref, td2_ref, slab_ref, out_ref):
    T, bt, _ = si_ref.shape

    w1 = slab_ref[_R_W1:_R_W1 + _IN, :_F0]
    w2 = slab_ref[_R_W2:_R_W2 + _F0, :_F1]
    w3 = slab_ref[_R_W3:_R_W3 + _F1, :_F2]
    w_tha = slab_ref[_R_WTH:_R_WTH + _N, :_NC1]
    w_thb = slab_ref[_R_WTH + _N:_R_WTH + 2 * _N, :_NC1]
    w_sp3 = slab_ref[_R_WSP3:_R_WSP3 + _F2, :_NC1]
    w_rd = slab_ref[_R_WRD:_R_WRD + _NC1, :_N]

    b1 = jnp.broadcast_to(slab_ref[_R_BIAS:_R_BIAS + 1, :_F0], (bt, _F0))
    b2 = jnp.broadcast_to(slab_ref[_R_BIAS + 1:_R_BIAS + 2, :_F1], (bt, _F1))
    bc1 = jnp.broadcast_to(slab_ref[_R_BIAS + 2:_R_BIAS + 3, :_NC1], (bt, _NC1))
    b3 = jnp.broadcast_to(slab_ref[_R_BIAS + 3:_R_BIAS + 4, :_F2], (bt, _F2))
    b_rd = jnp.broadcast_to(slab_ref[_R_BIAS + 4:_R_BIAS + 5, :_N], (bt, _N))

    td_flat = td_ref[...].reshape(T * bt, _N)
    td2_flat = td2_ref[...].reshape(T * bt, _N)
    th_c = (jnp.dot(td_flat, w_tha, preferred_element_type=jnp.float32)
            + jnp.dot(td2_flat, w_thb, preferred_element_type=jnp.float32))

    def lif(cur, mem_old):
        mem_new = (_BETA * mem_old + cur
                   - jnp.where(mem_old > _THR, _THR, 0.0))
        return jnp.where(mem_new > _THR, 1.0, 0.0), mem_new

    mem1 = jnp.zeros((bt, _F0), jnp.float32)
    mem2 = jnp.zeros((bt, _F1), jnp.float32)
    mem3 = jnp.zeros((bt, _F2), jnp.float32)
    mem4 = jnp.zeros((bt, _NC1), jnp.float32)
    spk4 = mem4

    for t in range(T):
        x0 = si_ref[t]
        spk1, mem1 = lif(
            jnp.dot(x0, w1, preferred_element_type=jnp.float32) + b1, mem1)
        spk2, mem2 = lif(
            jnp.dot(spk1, w2, preferred_element_type=jnp.float32) + b2, mem2)
        spk3, mem3 = lif(
            jnp.dot(spk2, w3, preferred_element_type=jnp.float32) + b3, mem3)
        cur4 = (th_c[t * bt:(t + 1) * bt, :] + bc1
                + jnp.dot(spk3, w_sp3, preferred_element_type=jnp.float32))
        spk4, mem4 = lif(cur4, mem4)

    out_ref[...] = jnp.dot(spk4, w_rd, preferred_element_type=jnp.float32) + b_rd


@functools.partial(jax.jit, static_argnames=("batch_tile",))
def _snn_forward(spk_in, spk_thdot, spk_thdot2, slab, batch_tile=512):
    T, B, _ = spk_in.shape
    bt = batch_tile if (B % batch_tile == 0) else B
    grid = (B // bt,)

    out = pl.pallas_call(
        _snn_body,
        out_shape=jax.ShapeDtypeStruct((B, _N), jnp.float32),
        grid=grid,
        in_specs=[
            pl.BlockSpec((T, bt, _IN), lambda b: (0, b, 0)),
            pl.BlockSpec((T, bt, _N), lambda b: (0, b, 0)),
            pl.BlockSpec((T, bt, _N), lambda b: (0, b, 0)),
            pl.BlockSpec((_SLAB_ROWS, _SLAB_W), lambda b: (0, 0)),
        ],
        out_specs=pl.BlockSpec((bt, _N), lambda b: (b, 0)),
        compiler_params=pltpu.CompilerParams(
            dimension_semantics=("parallel",)),
    )(spk_in, spk_thdot, spk_thdot2, slab)
    return out[:, :_OUT]


def kernel(spk_in, spk_thdot, spk_thdot2, slab):
    return _snn_forward(spk_in, spk_thdot, spk_thdot2, slab)


# speedup vs baseline: 2.8014x; 1.8447x over previous
import functools

import jax
import jax.numpy as jnp
from jax.experimental import pallas as pl
from jax.experimental.pallas import tpu as pltpu

_T = 8
_N = 8
_F0 = 32
_F1 = 32
_F2 = _N
_NC1 = 32
_IN = 2 * _N
_OUT = 3
_BETA = 0.9
_P = 4
_W = 32
_LANES = _P * _W

def _al8(r):
    return (r + 7) // 8 * 8

_R_W1 = 0
_R_W2 = _al8(_R_W1 + _IN)
_R_WTH = _al8(_R_W2 + _F0)
_R_WSP3 = _al8(_R_WTH + 2 * _N)
_R_W3 = _al8(_R_WSP3 + _F2)
_R_WRD = _al8(_R_W3 + _F1)
_R_BIAS = _al8(_R_WRD + _NC1)

_L_W1, _L_WTH, _L_W2, _L_W3, _L_WSP, _L_WRD = (i * _LANES for i in range(6))
_L_B = 6 * _LANES
_XSLAB_ROWS = _al8(_L_B + 5)


def _expand_weights(slab):
    z = jnp.zeros((_W, _W), jnp.float32)
    w1 = z.at[:_IN, :].set(slab[_R_W1:_R_W1 + _IN, :_F0])
    wth = z.at[_IN:_IN + 2 * _N, :].set(slab[_R_WTH:_R_WTH + 2 * _N, :_NC1])
    w2 = slab[_R_W2:_R_W2 + _F0, :_F1]
    w3 = z.at[:, :_N].set(slab[_R_W3:_R_W3 + _F1, :_F2])
    wsp = z.at[:_N, :].set(slab[_R_WSP3:_R_WSP3 + _F2, :_NC1])
    wrd = z.at[:, :_N].set(slab[_R_WRD:_R_WRD + _NC1, :_N])
    mats = jnp.stack([w1, wth, w2, w3, wsp, wrd])
    eye = jnp.eye(_P, dtype=jnp.float32)
    big = (eye[:, None, :, None] * mats[:, None, :, None, :])
    big = big.reshape(6 * _LANES, _LANES)

    zb = jnp.zeros((1, _W), jnp.float32)
    b1 = slab[_R_BIAS:_R_BIAS + 1, :_F0]
    b2 = slab[_R_BIAS + 1:_R_BIAS + 2, :_F1]
    bc1 = slab[_R_BIAS + 2:_R_BIAS + 3, :_NC1]
    b3 = zb.at[:, :_N].set(slab[_R_BIAS + 3:_R_BIAS + 4, :_F2])
    brd = zb.at[:, :_N].set(slab[_R_BIAS + 4:_R_BIAS + 5, :_N])
    biases = jnp.tile(jnp.concatenate([b1, b2, bc1, b3, brd], axis=0),
                      (1, _P))
    pad = jnp.zeros((_XSLAB_ROWS - _L_B - 5, _LANES), jnp.float32)
    return jnp.concatenate([big, biases, pad], axis=0)


def _snn_body(x_ref, w_ref, out_ref):
    T = x_ref.shape[0]

    w1 = w_ref[_L_W1:_L_W1 + _LANES, :]
    wth = w_ref[_L_WTH:_L_WTH + _LANES, :]
    w2 = w_ref[_L_W2:_L_W2 + _LANES, :]
    w3 = w_ref[_L_W3:_L_W3 + _LANES, :]
    wsp = w_ref[_L_WSP:_L_WSP + _LANES, :]
    wrd = w_ref[_L_WRD:_L_WRD + _LANES, :]
    b1 = w_ref[_L_B:_L_B + 1, :]
    b2 = w_ref[_L_B + 1:_L_B + 2, :]
    bc1 = w_ref[_L_B + 2:_L_B + 3, :]
    b3 = w_ref[_L_B + 3:_L_B + 4, :]
    brd = w_ref[_L_B + 4:_L_B + 5, :]

    rows = x_ref.shape[1]
    zero = jnp.zeros((rows, _LANES), jnp.float32)
    mem1 = mem2 = mem3 = mem4 = zero
    spk1 = spk2 = spk3 = spk4 = zero

    def lif(cur, mem_old, spk_prev):
        mem_new = _BETA * mem_old + cur - spk_prev
        spk = jnp.where(mem_new > 1.0, 1.0, 0.0)
        return spk, mem_new

    for t in range(T):
        x = x_ref[t]
        spk1, mem1 = lif(
            jnp.dot(x, w1, preferred_element_type=jnp.float32) + b1,
            mem1, spk1)
        spk2, mem2 = lif(
            jnp.dot(spk1, w2, preferred_element_type=jnp.float32) + b2,
            mem2, spk2)
        spk3, mem3 = lif(
            jnp.dot(spk2, w3, preferred_element_type=jnp.float32) + b3,
            mem3, spk3)
        cur4 = (jnp.dot(x, wth, preferred_element_type=jnp.float32)
                + jnp.dot(spk3, wsp, preferred_element_type=jnp.float32)
                + bc1)
        spk4, mem4 = lif(cur4, mem4, spk4)

    out_ref[...] = jnp.dot(spk4, wrd, preferred_element_type=jnp.float32) + brd


@functools.partial(jax.jit, static_argnames=("batch_tile",))
def _snn_forward(spk_in, spk_thdot, spk_thdot2, slab, batch_tile=2048):
    T, B, _ = spk_in.shape
    bt = batch_tile if (B % batch_tile == 0) else B
    rows = bt // _P

    packed = jnp.concatenate(
        [spk_in, spk_thdot, spk_thdot2], axis=-1).reshape(T, B // _P, _LANES)
    wslab = _expand_weights(slab)

    out = pl.pallas_call(
        _snn_body,
        out_shape=jax.ShapeDtypeStruct((B // _P, _LANES), jnp.float32),
        grid=(B // bt,),
        in_specs=[
            pl.BlockSpec((T, rows, _LANES), lambda b: (0, b, 0)),
            pl.BlockSpec((_XSLAB_ROWS, _LANES), lambda b: (0, 0)),
        ],
        out_specs=pl.BlockSpec((rows, _LANES), lambda b: (b, 0)),
        compiler_params=pltpu.CompilerParams(
            dimension_semantics=("parallel",)),
    )(packed, wslab)
    return out.reshape(B, _W)[:, :_OUT]


def kernel(spk_in, spk_thdot, spk_thdot2, slab):
    return _snn_forward(spk_in, spk_thdot, spk_thdot2, slab)


# speedup vs baseline: 3.1271x; 1.1163x over previous
import functools

import jax
import jax.numpy as jnp
from jax.experimental import pallas as pl
from jax.experimental.pallas import tpu as pltpu

_T = 8
_N = 8
_F0 = 32
_F1 = 32
_F2 = _N
_NC1 = 32
_IN = 2 * _N
_OUT = 3
_BETA = 0.9
_P = 4
_W = 32
_LANES = _P * _W

def _al8(r):
    return (r + 7) // 8 * 8

_R_W1 = 0
_R_W2 = _al8(_R_W1 + _IN)
_R_WTH = _al8(_R_W2 + _F0)
_R_WSP3 = _al8(_R_WTH + 2 * _N)
_R_W3 = _al8(_R_WSP3 + _F2)
_R_WRD = _al8(_R_W3 + _F1)
_R_BIAS = _al8(_R_WRD + _NC1)

_L_W1, _L_WTH, _L_W2, _L_W3, _L_WSP, _L_WRD = (i * _LANES for i in range(6))
_L_B = 6 * _LANES
_XSLAB_ROWS = _al8(_L_B + 5)


def _expand_weights(slab):
    z = jnp.zeros((_W, _W), jnp.float32)
    w1 = z.at[:_IN, :].set(slab[_R_W1:_R_W1 + _IN, :_F0])
    wth = z.at[_IN:_IN + 2 * _N, :].set(slab[_R_WTH:_R_WTH + 2 * _N, :_NC1])
    w2 = slab[_R_W2:_R_W2 + _F0, :_F1]
    w3 = z.at[:, :_N].set(slab[_R_W3:_R_W3 + _F1, :_F2])
    wsp = z.at[:_N, :].set(slab[_R_WSP3:_R_WSP3 + _F2, :_NC1])
    wrd = z.at[:, :_N].set(slab[_R_WRD:_R_WRD + _NC1, :_N])
    mats = jnp.stack([w1, wth, w2, w3, wsp, wrd])
    eye = jnp.eye(_P, dtype=jnp.float32)
    big = (eye[:, None, :, None] * mats[:, None, :, None, :])
    big = big.reshape(6 * _LANES, _LANES)

    zb = jnp.zeros((1, _W), jnp.float32)
    b1 = slab[_R_BIAS:_R_BIAS + 1, :_F0]
    b2 = slab[_R_BIAS + 1:_R_BIAS + 2, :_F1]
    bc1 = slab[_R_BIAS + 2:_R_BIAS + 3, :_NC1]
    b3 = zb.at[:, :_N].set(slab[_R_BIAS + 3:_R_BIAS + 4, :_F2])
    brd = zb.at[:, :_N].set(slab[_R_BIAS + 4:_R_BIAS + 5, :_N])
    biases = jnp.tile(jnp.concatenate([b1, b2, bc1, b3, brd], axis=0),
                      (1, _P))
    pad = jnp.zeros((_XSLAB_ROWS - _L_B - 5, _LANES), jnp.float32)
    return jnp.concatenate([big, biases, pad], axis=0)


def _snn_body(x_ref, w_ref, out_ref):
    T = x_ref.shape[0]

    w1 = w_ref[_L_W1:_L_W1 + _LANES, :]
    wth = w_ref[_L_WTH:_L_WTH + _LANES, :]
    w2 = w_ref[_L_W2:_L_W2 + _LANES, :]
    w3 = w_ref[_L_W3:_L_W3 + _LANES, :]
    wsp = w_ref[_L_WSP:_L_WSP + _LANES, :]
    wrd = w_ref[_L_WRD:_L_WRD + _LANES, :]
    b1 = w_ref[_L_B:_L_B + 1, :]
    b2 = w_ref[_L_B + 1:_L_B + 2, :]
    bc1 = w_ref[_L_B + 2:_L_B + 3, :]
    b3 = w_ref[_L_B + 3:_L_B + 4, :]
    brd = w_ref[_L_B + 4:_L_B + 5, :]

    rows = x_ref.shape[1]
    zero = jnp.zeros((rows, _LANES), jnp.float32)
    mem1 = mem2 = mem3 = mem4 = zero
    spk1 = spk2 = spk3 = spk4 = zero

    def lif(cur, mem_old, spk_prev):
        mem_new = _BETA * mem_old + cur - spk_prev
        spk = jnp.where(mem_new > 1.0, 1.0, 0.0)
        return spk, mem_new

    for t in range(T):
        x = x_ref[t].astype(jnp.float32)
        spk1, mem1 = lif(
            jnp.dot(x, w1, preferred_element_type=jnp.float32) + b1,
            mem1, spk1)
        spk2, mem2 = lif(
            jnp.dot(spk1, w2, preferred_element_type=jnp.float32) + b2,
            mem2, spk2)
        spk3, mem3 = lif(
            jnp.dot(spk2, w3, preferred_element_type=jnp.float32) + b3,
            mem3, spk3)
        cur4 = (jnp.dot(x, wth, preferred_element_type=jnp.float32)
                + jnp.dot(spk3, wsp, preferred_element_type=jnp.float32)
                + bc1)
        spk4, mem4 = lif(cur4, mem4, spk4)

    out_ref[...] = jnp.dot(spk4, wrd, preferred_element_type=jnp.float32) + brd


@functools.partial(jax.jit, static_argnames=("batch_tile",))
def _snn_forward(spk_in, spk_thdot, spk_thdot2, slab, batch_tile=2048):
    T, B, _ = spk_in.shape
    bt = batch_tile if (B % batch_tile == 0) else B
    rows = bt // _P

    packed = jnp.concatenate(
        [spk_in, spk_thdot, spk_thdot2],
        axis=-1).astype(jnp.bfloat16).reshape(T, B // _P, _LANES)
    wslab = _expand_weights(slab)

    out = pl.pallas_call(
        _snn_body,
        out_shape=jax.ShapeDtypeStruct((B // _P, _LANES), jnp.float32),
        grid=(B // bt,),
        in_specs=[
            pl.BlockSpec((T, rows, _LANES), lambda b: (0, b, 0)),
            pl.BlockSpec((_XSLAB_ROWS, _LANES), lambda b: (0, 0)),
        ],
        out_specs=pl.BlockSpec((rows, _LANES), lambda b: (b, 0)),
        compiler_params=pltpu.CompilerParams(
            dimension_semantics=("parallel",)),
    )(packed, wslab)
    return out.reshape(B, _W)[:, :_OUT]


def kernel(spk_in, spk_thdot, spk_thdot2, slab):
    return _snn_forward(spk_in, spk_thdot, spk_thdot2, slab)


# speedup vs baseline: 3.3240x; 1.0629x over previous
import functools

import jax
import jax.numpy as jnp
from jax.experimental import pallas as pl
from jax.experimental.pallas import tpu as pltpu

_T = 8
_N = 8
_F0 = 32
_F1 = 32
_F2 = _N
_NC1 = 32
_IN = 2 * _N
_OUT = 3
_BETA = 0.9
_P = 4
_W = 32
_LANES = _P * _W

def _al8(r):
    return (r + 7) // 8 * 8

_R_W1 = 0
_R_W2 = _al8(_R_W1 + _IN)
_R_WTH = _al8(_R_W2 + _F0)
_R_WSP3 = _al8(_R_WTH + 2 * _N)
_R_W3 = _al8(_R_WSP3 + _F2)
_R_WRD = _al8(_R_W3 + _F1)
_R_BIAS = _al8(_R_WRD + _NC1)

_L_W1, _L_WTH, _L_W2, _L_W3, _L_WSP, _L_WRD = (i * _LANES for i in range(6))
_L_B = 6 * _LANES
_XSLAB_ROWS = _al8(_L_B + 5)


def _expand_weights(slab):
    z = jnp.zeros((_W, _W), jnp.float32)
    w1 = z.at[:_IN, :].set(slab[_R_W1:_R_W1 + _IN, :_F0])
    wth = z.at[_IN:_IN + 2 * _N, :].set(slab[_R_WTH:_R_WTH + 2 * _N, :_NC1])
    w2 = slab[_R_W2:_R_W2 + _F0, :_F1]
    w3 = z.at[:, :_N].set(slab[_R_W3:_R_W3 + _F1, :_F2])
    wsp = z.at[:_N, :].set(slab[_R_WSP3:_R_WSP3 + _F2, :_NC1])
    wrd = z.at[:, :_N].set(slab[_R_WRD:_R_WRD + _NC1, :_N])
    mats = jnp.stack([w1, wth, w2, w3, wsp, wrd])
    eye = jnp.eye(_P, dtype=jnp.float32)
    big = (eye[:, None, :, None] * mats[:, None, :, None, :])
    big = big.reshape(6 * _LANES, _LANES)

    zb = jnp.zeros((1, _W), jnp.float32)
    b1 = slab[_R_BIAS:_R_BIAS + 1, :_F0]
    b2 = slab[_R_BIAS + 1:_R_BIAS + 2, :_F1]
    bc1 = slab[_R_BIAS + 2:_R_BIAS + 3, :_NC1]
    b3 = zb.at[:, :_N].set(slab[_R_BIAS + 3:_R_BIAS + 4, :_F2])
    brd = zb.at[:, :_N].set(slab[_R_BIAS + 4:_R_BIAS + 5, :_N])
    biases = jnp.tile(jnp.concatenate([b1, b2, bc1, b3, brd], axis=0),
                      (1, _P))
    pad = jnp.zeros((_XSLAB_ROWS - _L_B - 5, _LANES), jnp.float32)
    return jnp.concatenate([big, biases, pad], axis=0)


def _snn_body(x_ref, w_ref, out_ref):
    T = x_ref.shape[0]

    w1 = w_ref[_L_W1:_L_W1 + _LANES, :]
    wth = w_ref[_L_WTH:_L_WTH + _LANES, :]
    w2 = w_ref[_L_W2:_L_W2 + _LANES, :]
    w3 = w_ref[_L_W3:_L_W3 + _LANES, :]
    wsp = w_ref[_L_WSP:_L_WSP + _LANES, :]
    wrd = w_ref[_L_WRD:_L_WRD + _LANES, :]
    b1 = w_ref[_L_B:_L_B + 1, :]
    b2 = w_ref[_L_B + 1:_L_B + 2, :]
    bc1 = w_ref[_L_B + 2:_L_B + 3, :]
    b3 = w_ref[_L_B + 3:_L_B + 4, :]
    brd = w_ref[_L_B + 4:_L_B + 5, :]

    rows = x_ref.shape[1]
    zero = jnp.zeros((rows, _LANES), jnp.float32)
    mem1 = mem2 = mem3 = mem4 = zero
    spk1 = spk2 = spk3 = spk4 = zero

    def lif(cur, mem_old, spk_prev):
        mem_new = _BETA * mem_old + cur - spk_prev
        spk = jnp.where(mem_new > 1.0, 1.0, 0.0)
        return spk, mem_new

    for t in range(T):
        x = x_ref[t].astype(jnp.float32)
        spk1, mem1 = lif(
            jnp.dot(x, w1, preferred_element_type=jnp.float32) + b1,
            mem1, spk1)
        spk2, mem2 = lif(
            jnp.dot(spk1, w2, preferred_element_type=jnp.float32) + b2,
            mem2, spk2)
        spk3, mem3 = lif(
            jnp.dot(spk2, w3, preferred_element_type=jnp.float32) + b3,
            mem3, spk3)
        cur4 = (jnp.dot(x, wth, preferred_element_type=jnp.float32)
                + jnp.dot(spk3, wsp, preferred_element_type=jnp.float32)
                + bc1)
        spk4, mem4 = lif(cur4, mem4, spk4)

    out_ref[...] = jnp.dot(spk4, wrd, preferred_element_type=jnp.float32) + brd


@functools.partial(jax.jit, static_argnames=("batch_tile",))
def _snn_forward(spk_in, spk_thdot, spk_thdot2, slab, batch_tile=4096):
    T, B, _ = spk_in.shape
    bt = batch_tile if (B % batch_tile == 0) else B
    rows = bt // _P

    packed = jnp.concatenate(
        [spk_in, spk_thdot, spk_thdot2],
        axis=-1).astype(jnp.bfloat16).reshape(T, B // _P, _LANES)
    wslab = _expand_weights(slab)

    out = pl.pallas_call(
        _snn_body,
        out_shape=jax.ShapeDtypeStruct((B // _P, _LANES), jnp.float32),
        grid=(B // bt,),
        in_specs=[
            pl.BlockSpec((T, rows, _LANES), lambda b: (0, b, 0)),
            pl.BlockSpec((_XSLAB_ROWS, _LANES), lambda b: (0, 0)),
        ],
        out_specs=pl.BlockSpec((rows, _LANES), lambda b: (b, 0)),
        compiler_params=pltpu.CompilerParams(
            dimension_semantics=("parallel",)),
    )(packed, wslab)
    return out.reshape(B, _W)[:, :_OUT]


def kernel(spk_in, spk_thdot, spk_thdot2, slab):
    return _snn_forward(spk_in, spk_thdot, spk_thdot2, slab)


# speedup vs baseline: 3.3279x; 1.0012x over previous
import functools

import jax
import jax.numpy as jnp
from jax.experimental import pallas as pl
from jax.experimental.pallas import tpu as pltpu

_T = 8
_N = 8
_F0 = 32
_F1 = 32
_F2 = _N
_NC1 = 32
_IN = 2 * _N
_OUT = 3
_BETA = 0.9
_P = 4
_W = 32
_LANES = _P * _W

def _al8(r):
    return (r + 7) // 8 * 8

_R_W1 = 0
_R_W2 = _al8(_R_W1 + _IN)
_R_WTH = _al8(_R_W2 + _F0)
_R_WSP3 = _al8(_R_WTH + 2 * _N)
_R_W3 = _al8(_R_WSP3 + _F2)
_R_WRD = _al8(_R_W3 + _F1)
_R_BIAS = _al8(_R_WRD + _NC1)

_L_W1, _L_WTH, _L_W2, _L_W3, _L_WSP, _L_WRD = (i * _LANES for i in range(6))
_L_B = 6 * _LANES
_XSLAB_ROWS = _al8(_L_B + 5)


def _expand_weights(slab):
    z = jnp.zeros((_W, _W), jnp.float32)
    w1 = z.at[:_IN, :].set(slab[_R_W1:_R_W1 + _IN, :_F0])
    wth = z.at[_IN:_IN + 2 * _N, :].set(slab[_R_WTH:_R_WTH + 2 * _N, :_NC1])
    w2 = slab[_R_W2:_R_W2 + _F0, :_F1]
    w3 = z.at[:, :_N].set(slab[_R_W3:_R_W3 + _F1, :_F2])
    wsp = z.at[:_N, :].set(slab[_R_WSP3:_R_WSP3 + _F2, :_NC1])
    wrd = z.at[:, :_N].set(slab[_R_WRD:_R_WRD + _NC1, :_N])
    mats = jnp.stack([w1, wth, w2, w3, wsp, wrd])
    eye = jnp.eye(_P, dtype=jnp.float32)
    big = (eye[:, None, :, None] * mats[:, None, :, None, :])
    big = big.reshape(6 * _LANES, _LANES)

    zb = jnp.zeros((1, _W), jnp.float32)
    b1 = slab[_R_BIAS:_R_BIAS + 1, :_F0]
    b2 = slab[_R_BIAS + 1:_R_BIAS + 2, :_F1]
    bc1 = slab[_R_BIAS + 2:_R_BIAS + 3, :_NC1]
    b3 = zb.at[:, :_N].set(slab[_R_BIAS + 3:_R_BIAS + 4, :_F2])
    brd = zb.at[:, :_N].set(slab[_R_BIAS + 4:_R_BIAS + 5, :_N])
    biases = jnp.tile(jnp.concatenate([b1, b2, bc1, b3, brd], axis=0),
                      (1, _P))
    pad = jnp.zeros((_XSLAB_ROWS - _L_B - 5, _LANES), jnp.float32)
    return jnp.concatenate([big, biases, pad], axis=0)


def _snn_body(x_ref, w_ref, out_ref):
    T = x_ref.shape[0]

    w1 = w_ref[_L_W1:_L_W1 + _LANES, :]
    wth = w_ref[_L_WTH:_L_WTH + _LANES, :]
    w2 = w_ref[_L_W2:_L_W2 + _LANES, :]
    w3 = w_ref[_L_W3:_L_W3 + _LANES, :]
    wsp = w_ref[_L_WSP:_L_WSP + _LANES, :]
    wrd = w_ref[_L_WRD:_L_WRD + _LANES, :]
    b1 = w_ref[_L_B:_L_B + 1, :]
    b2 = w_ref[_L_B + 1:_L_B + 2, :]
    bc1 = w_ref[_L_B + 2:_L_B + 3, :]
    b3 = w_ref[_L_B + 3:_L_B + 4, :]
    brd = w_ref[_L_B + 4:_L_B + 5, :]

    rows = x_ref.shape[1]
    zero = jnp.zeros((rows, _LANES), jnp.float32)
    mem1 = mem2 = mem3 = mem4 = zero
    spk1 = spk2 = spk3 = spk4 = zero

    def lif(cur, mem_old, spk_prev):
        mem_new = _BETA * mem_old + cur - spk_prev
        spk = jnp.where(mem_new > 1.0, 1.0, 0.0)
        return spk, mem_new

    for t in range(T):
        x = x_ref[t].astype(jnp.float32)
        spk1, mem1 = lif(
            jnp.dot(x, w1, preferred_element_type=jnp.float32) + b1,
            mem1, spk1)
        spk2, mem2 = lif(
            jnp.dot(spk1, w2, preferred_element_type=jnp.float32) + b2,
            mem2, spk2)
        spk3, mem3 = lif(
            jnp.dot(spk2, w3, preferred_element_type=jnp.float32) + b3,
            mem3, spk3)
        cur4 = (jnp.dot(x, wth, preferred_element_type=jnp.float32)
                + jnp.dot(spk3, wsp, preferred_element_type=jnp.float32)
                + bc1)
        spk4, mem4 = lif(cur4, mem4, spk4)

    out_ref[...] = jnp.dot(spk4, wrd, preferred_element_type=jnp.float32) + brd


@functools.partial(jax.jit, static_argnames=("batch_tile",))
def _snn_forward(spk_in, spk_thdot, spk_thdot2, slab, batch_tile=8192):
    T, B, _ = spk_in.shape
    bt = batch_tile if (B % batch_tile == 0) else B
    rows = bt // _P

    packed = jnp.concatenate(
        [spk_in, spk_thdot, spk_thdot2],
        axis=-1).astype(jnp.bfloat16).reshape(T, B // _P, _LANES)
    wslab = _expand_weights(slab)

    out = pl.pallas_call(
        _snn_body,
        out_shape=jax.ShapeDtypeStruct((B // _P, _LANES), jnp.float32),
        grid=(B // bt,),
        in_specs=[
            pl.BlockSpec((T, rows, _LANES), lambda b: (0, b, 0)),
            pl.BlockSpec((_XSLAB_ROWS, _LANES), lambda b: (0, 0)),
        ],
        out_specs=pl.BlockSpec((rows, _LANES), lambda b: (b, 0)),
        compiler_params=pltpu.CompilerParams(
            dimension_semantics=("parallel",)),
    )(packed, wslab)
    return out.reshape(B, _W)[:, :_OUT]


def kernel(spk_in, spk_thdot, spk_thdot2, slab):
    return _snn_forward(spk_in, spk_thdot, spk_thdot2, slab)


# speedup vs baseline: 3.3298x; 1.0006x over previous
import functools

import jax
import jax.numpy as jnp
from jax.experimental import pallas as pl
from jax.experimental.pallas import tpu as pltpu

_T = 8
_N = 8
_F0 = 32
_F1 = 32
_F2 = _N
_NC1 = 32
_IN = 2 * _N
_OUT = 3
_BETA = 0.9
_P = 4
_W = 32
_LANES = _P * _W

def _al8(r):
    return (r + 7) // 8 * 8

_R_W1 = 0
_R_W2 = _al8(_R_W1 + _IN)
_R_WTH = _al8(_R_W2 + _F0)
_R_WSP3 = _al8(_R_WTH + 2 * _N)
_R_W3 = _al8(_R_WSP3 + _F2)
_R_WRD = _al8(_R_W3 + _F1)
_R_BIAS = _al8(_R_WRD + _NC1)

_L_W1, _L_WTH, _L_W2, _L_W3, _L_WSP, _L_WRD = (i * _LANES for i in range(6))
_L_B = 6 * _LANES
_XSLAB_ROWS = _al8(_L_B + 5)


def _expand_weights(slab):
    z = jnp.zeros((_W, _W), jnp.float32)
    w1 = z.at[:_IN, :].set(slab[_R_W1:_R_W1 + _IN, :_F0])
    wth = z.at[_IN:_IN + 2 * _N, :].set(slab[_R_WTH:_R_WTH + 2 * _N, :_NC1])
    w2 = slab[_R_W2:_R_W2 + _F0, :_F1]
    w3 = z.at[:, :_N].set(slab[_R_W3:_R_W3 + _F1, :_F2])
    wsp = z.at[:_N, :].set(slab[_R_WSP3:_R_WSP3 + _F2, :_NC1])
    mats = jnp.stack([w1, wth, w2, w3, wsp])
    eye = jnp.eye(_P, dtype=jnp.float32)
    big = (eye[:, None, :, None] * mats[:, None, :, None, :])
    big = big.reshape(5 * _LANES, _LANES)
    wrd_r = slab[_R_WRD:_R_WRD + _NC1, :_N]
    wrd = (eye[:, None, :, None] * wrd_r[None, :, None, :]).reshape(
        _LANES, _P * _N)
    wrd = jnp.pad(wrd, ((0, 0), (0, _LANES - _P * _N)))
    big = jnp.concatenate([big, wrd], axis=0)

    zb = jnp.zeros((1, _W), jnp.float32)
    b1 = slab[_R_BIAS:_R_BIAS + 1, :_F0]
    b2 = slab[_R_BIAS + 1:_R_BIAS + 2, :_F1]
    bc1 = slab[_R_BIAS + 2:_R_BIAS + 3, :_NC1]
    b3 = zb.at[:, :_N].set(slab[_R_BIAS + 3:_R_BIAS + 4, :_F2])
    biases = jnp.tile(jnp.concatenate([b1, b2, bc1, b3], axis=0),
                      (1, _P))
    brd = jnp.pad(jnp.tile(slab[_R_BIAS + 4:_R_BIAS + 5, :_N], (1, _P)),
                  ((0, 0), (0, _LANES - _P * _N)))
    biases = jnp.concatenate([biases, brd], axis=0)
    pad = jnp.zeros((_XSLAB_ROWS - _L_B - 5, _LANES), jnp.float32)
    return jnp.concatenate([big, biases, pad], axis=0)


def _snn_body(x_ref, w_ref, out_ref):
    T = x_ref.shape[0]

    w1 = w_ref[_L_W1:_L_W1 + _LANES, :]
    wth = w_ref[_L_WTH:_L_WTH + _LANES, :]
    w2 = w_ref[_L_W2:_L_W2 + _LANES, :]
    w3 = w_ref[_L_W3:_L_W3 + _LANES, :]
    wsp = w_ref[_L_WSP:_L_WSP + _LANES, :]
    wrd = w_ref[_L_WRD:_L_WRD + _LANES, :_P * _N]
    b1 = w_ref[_L_B:_L_B + 1, :]
    b2 = w_ref[_L_B + 1:_L_B + 2, :]
    bc1 = w_ref[_L_B + 2:_L_B + 3, :]
    b3 = w_ref[_L_B + 3:_L_B + 4, :]
    brd = w_ref[_L_B + 4:_L_B + 5, :_P * _N]

    rows = x_ref.shape[1]
    zero = jnp.zeros((rows, _LANES), jnp.float32)
    mem1 = mem2 = mem3 = mem4 = zero
    spk1 = spk2 = spk3 = spk4 = zero

    def lif(cur, mem_old, spk_prev):
        mem_new = _BETA * mem_old + cur - spk_prev
        spk = jnp.where(mem_new > 1.0, 1.0, 0.0)
        return spk, mem_new

    for t in range(T):
        x = x_ref[t].astype(jnp.float32)
        spk1, mem1 = lif(
            jnp.dot(x, w1, preferred_element_type=jnp.float32) + b1,
            mem1, spk1)
        spk2, mem2 = lif(
            jnp.dot(spk1, w2, preferred_element_type=jnp.float32) + b2,
            mem2, spk2)
        spk3, mem3 = lif(
            jnp.dot(spk2, w3, preferred_element_type=jnp.float32) + b3,
            mem3, spk3)
        cur4 = (jnp.dot(x, wth, preferred_element_type=jnp.float32)
                + jnp.dot(spk3, wsp, preferred_element_type=jnp.float32)
                + bc1)
        spk4, mem4 = lif(cur4, mem4, spk4)

    out_ref[...] = jnp.dot(spk4, wrd, preferred_element_type=jnp.float32) + brd


@functools.partial(jax.jit, static_argnames=("batch_tile",))
def _snn_forward(spk_in, spk_thdot, spk_thdot2, slab, batch_tile=8192):
    T, B, _ = spk_in.shape
    bt = batch_tile if (B % batch_tile == 0) else B
    rows = bt // _P

    packed = jnp.concatenate(
        [spk_in, spk_thdot, spk_thdot2],
        axis=-1).astype(jnp.bfloat16).reshape(T, B // _P, _LANES)
    wslab = _expand_weights(slab)

    out = pl.pallas_call(
        _snn_body,
        out_shape=jax.ShapeDtypeStruct((B // _P, _P * _N), jnp.float32),
        grid=(B // bt,),
        in_specs=[
            pl.BlockSpec((T, rows, _LANES), lambda b: (0, b, 0)),
            pl.BlockSpec((_XSLAB_ROWS, _LANES), lambda b: (0, 0)),
        ],
        out_specs=pl.BlockSpec((rows, _P * _N), lambda b: (b, 0)),
        compiler_params=pltpu.CompilerParams(
            dimension_semantics=("parallel",)),
    )(packed, wslab)
    return out.reshape(B, _N)[:, :_OUT]


def kernel(spk_in, spk_thdot, spk_thdot2, slab):
    return _snn_forward(spk_in, spk_thdot, spk_thdot2, slab)
